# baseline (device time: 27769 ns/iter reference)
import jax
import jax.numpy as jnp
from jax import lax
from jax.experimental import pallas as pl
from jax.experimental.pallas import tpu as pltpu

BM = 512


def kernel(x, dy, gamma):
    m, d = x.shape
    n_blocks = m // BM

    def body(x_ref, dy_ref, out_ref, acc_ref, recv_ref, send_sem, recv_sem):
        i = pl.program_id(0)

        @pl.when(i == 0)
        def _():
            acc_ref[...] = jnp.zeros_like(acc_ref)

        xb = x_ref[...]
        dyb = dy_ref[...]
        mu = jnp.mean(xb, axis=1, keepdims=True)
        diff = xb - mu
        var = jnp.mean(diff * diff, axis=1, keepdims=True)
        xhat = diff * lax.rsqrt(var + 1e-5)
        pdg = jnp.sum(dyb * xhat, axis=0)[None, :]
        pdb = jnp.sum(dyb, axis=0)[None, :]
        acc_ref[...] += jnp.concatenate([pdg, pdb], axis=0)

        @pl.when(i == n_blocks - 1)
        def _():
            my_x = lax.axis_index("x")
            my_y = lax.axis_index("y")
            my_z = lax.axis_index("z")
            partner = (1 - my_x, my_y, my_z)

            barrier = pltpu.get_barrier_semaphore()
            pl.semaphore_signal(
                barrier, inc=1,
                device_id=partner, device_id_type=pl.DeviceIdType.MESH,
            )
            pl.semaphore_wait(barrier, 1)

            rdma = pltpu.make_async_remote_copy(
                src_ref=acc_ref,
                dst_ref=recv_ref,
                send_sem=send_sem,
                recv_sem=recv_sem,
                device_id=partner,
                device_id_type=pl.DeviceIdType.MESH,
            )
            rdma.start()
            rdma.wait()

            out_ref[...] = acc_ref[...] + recv_ref[...]

    return pl.pallas_call(
        body,
        grid=(n_blocks,),
        in_specs=[
            pl.BlockSpec((BM, d), lambda i: (i, 0)),
            pl.BlockSpec((BM, d), lambda i: (i, 0)),
        ],
        out_specs=pl.BlockSpec((2, d), lambda i: (0, 0)),
        out_shape=jax.ShapeDtypeStruct((2, d), jnp.float32),
        scratch_shapes=[
            pltpu.VMEM((2, d), jnp.float32),
            pltpu.VMEM((2, d), jnp.float32),
            pltpu.SemaphoreType.DMA,
            pltpu.SemaphoreType.DMA,
        ],
        compiler_params=pltpu.CompilerParams(collective_id=0),
    )(x, dy)


# device time: 14817 ns/iter; 1.8741x vs baseline; 1.8741x over previous
import jax
import jax.numpy as jnp
from jax import lax
from jax.experimental import pallas as pl
from jax.experimental.pallas import tpu as pltpu

N_DEV = 8
BM = 256
QROWS = 1024


def kernel(x, dy, gamma):
    m, d = x.shape
    n_blocks = QROWS // BM

    def body(x_ref, dy_ref, out_ref, acc_ref, recv_ref, send_sems, recv_sems):
        i = pl.program_id(0)
        my_id = (
            lax.axis_index("x") * 4 + lax.axis_index("y") * 2 + lax.axis_index("z")
        )

        @pl.when(i == 0)
        def _():
            barrier = pltpu.get_barrier_semaphore()
            for k in range(N_DEV):
                @pl.when(my_id != k)
                def _():
                    pl.semaphore_signal(
                        barrier, inc=1,
                        device_id=k, device_id_type=pl.DeviceIdType.LOGICAL,
                    )
            pl.semaphore_wait(barrier, N_DEV - 1)

        @pl.when(i == 0)
        def _():
            acc_ref[...] = jnp.zeros_like(acc_ref)

        xb = x_ref[...]
        dyb = dy_ref[...]
        mu = jnp.mean(xb, axis=1, keepdims=True)
        diff = xb - mu
        var = jnp.mean(diff * diff, axis=1, keepdims=True)
        xhat = diff * lax.rsqrt(var + 1e-5)
        pdg = jnp.sum(dyb * xhat, axis=0)[None, :]
        pdb = jnp.sum(dyb, axis=0)[None, :]
        acc_ref[...] += jnp.concatenate([pdg, pdb], axis=0)

        @pl.when(i == n_blocks - 1)
        def _():
            send_descs = [
                pltpu.make_async_remote_copy(
                    src_ref=acc_ref,
                    dst_ref=recv_ref.at[my_id],
                    send_sem=send_sems.at[k],
                    recv_sem=recv_sems.at[my_id],
                    device_id=k,
                    device_id_type=pl.DeviceIdType.LOGICAL,
                )
                for k in range(N_DEV)
            ]
            recv_descs = [
                pltpu.make_async_remote_copy(
                    src_ref=acc_ref,
                    dst_ref=recv_ref.at[k],
                    send_sem=send_sems.at[k],
                    recv_sem=recv_sems.at[k],
                    device_id=k,
                    device_id_type=pl.DeviceIdType.LOGICAL,
                )
                for k in range(N_DEV)
            ]

            for k in range(N_DEV):
                @pl.when(my_id != k)
                def _(k=k):
                    send_descs[k].start()

            total = acc_ref[...]
            for k in range(N_DEV):
                @pl.when(my_id != k)
                def _(k=k):
                    recv_descs[k].wait_recv()
                total = total + jnp.where(my_id != k, recv_ref[k], 0.0)
            out_ref[...] = total

            for k in range(N_DEV):
                @pl.when(my_id != k)
                def _(k=k):
                    send_descs[k].wait_send()

    def row_block(i):
        r = lax.axis_index("y") * 2 + lax.axis_index("z")
        return (r * n_blocks + i, 0)

    return pl.pallas_call(
        body,
        grid=(n_blocks,),
        in_specs=[
            pl.BlockSpec((BM, d), row_block),
            pl.BlockSpec((BM, d), row_block),
        ],
        out_specs=pl.BlockSpec((2, d), lambda i: (0, 0)),
        out_shape=jax.ShapeDtypeStruct((2, d), jnp.float32),
        scratch_shapes=[
            pltpu.VMEM((2, d), jnp.float32),
            pltpu.VMEM((N_DEV, 2, d), jnp.float32),
            pltpu.SemaphoreType.DMA((N_DEV,)),
            pltpu.SemaphoreType.DMA((N_DEV,)),
        ],
        compiler_params=pltpu.CompilerParams(collective_id=0),
    )(x, dy)


# device time: 14273 ns/iter; 1.9456x vs baseline; 1.0381x over previous
import jax
import jax.numpy as jnp
from jax import lax
from jax.experimental import pallas as pl
from jax.experimental.pallas import tpu as pltpu

N_DEV = 8
BM = 512
QROWS = 1024


def kernel(x, dy, gamma):
    m, d = x.shape
    n_blocks = QROWS // BM

    def body(x_ref, dy_ref, out_ref, acc_ref, recv_ref, send_sems, recv_sems):
        i = pl.program_id(0)
        my_id = (
            lax.axis_index("x") * 4 + lax.axis_index("y") * 2 + lax.axis_index("z")
        )

        @pl.when(i == 0)
        def _():
            barrier = pltpu.get_barrier_semaphore()
            for k in range(N_DEV):
                @pl.when(my_id != k)
                def _():
                    pl.semaphore_signal(
                        barrier, inc=1,
                        device_id=k, device_id_type=pl.DeviceIdType.LOGICAL,
                    )
            pl.semaphore_wait(barrier, N_DEV - 1)

        @pl.when(i == 0)
        def _():
            acc_ref[...] = jnp.zeros_like(acc_ref)

        xb = x_ref[...]
        dyb = dy_ref[...]
        mu = jnp.mean(xb, axis=1, keepdims=True)
        diff = xb - mu
        var = jnp.mean(diff * diff, axis=1, keepdims=True)
        xhat = diff * lax.rsqrt(var + 1e-5)
        pdg = jnp.sum(dyb * xhat, axis=0)[None, :]
        pdb = jnp.sum(dyb, axis=0)[None, :]
        acc_ref[...] += jnp.concatenate([pdg, pdb], axis=0)

        @pl.when(i == n_blocks - 1)
        def _():
            send_descs = [
                pltpu.make_async_remote_copy(
                    src_ref=acc_ref,
                    dst_ref=recv_ref.at[my_id],
                    send_sem=send_sems.at[k],
                    recv_sem=recv_sems.at[my_id],
                    device_id=k,
                    device_id_type=pl.DeviceIdType.LOGICAL,
                )
                for k in range(N_DEV)
            ]
            recv_descs = [
                pltpu.make_async_remote_copy(
                    src_ref=acc_ref,
                    dst_ref=recv_ref.at[k],
                    send_sem=send_sems.at[k],
                    recv_sem=recv_sems.at[k],
                    device_id=k,
                    device_id_type=pl.DeviceIdType.LOGICAL,
                )
                for k in range(N_DEV)
            ]

            for k in range(N_DEV):
                @pl.when(my_id != k)
                def _(k=k):
                    send_descs[k].start()

            total = acc_ref[...]
            for k in range(N_DEV):
                @pl.when(my_id != k)
                def _(k=k):
                    recv_descs[k].wait_recv()
                total = total + jnp.where(my_id != k, recv_ref[k], 0.0)
            out_ref[...] = total

            for k in range(N_DEV):
                @pl.when(my_id != k)
                def _(k=k):
                    send_descs[k].wait_send()

    def row_block(i):
        r = lax.axis_index("y") * 2 + lax.axis_index("z")
        return (r * n_blocks + i, 0)

    return pl.pallas_call(
        body,
        grid=(n_blocks,),
        in_specs=[
            pl.BlockSpec((BM, d), row_block),
            pl.BlockSpec((BM, d), row_block),
        ],
        out_specs=pl.BlockSpec((2, d), lambda i: (0, 0)),
        out_shape=jax.ShapeDtypeStruct((2, d), jnp.float32),
        scratch_shapes=[
            pltpu.VMEM((2, d), jnp.float32),
            pltpu.VMEM((N_DEV, 2, d), jnp.float32),
            pltpu.SemaphoreType.DMA((N_DEV,)),
            pltpu.SemaphoreType.DMA((N_DEV,)),
        ],
        compiler_params=pltpu.CompilerParams(collective_id=0),
    )(x, dy)


# device time: 13523 ns/iter; 2.0535x vs baseline; 1.0555x over previous
import jax
import jax.numpy as jnp
from jax import lax
from jax.experimental import pallas as pl
from jax.experimental.pallas import tpu as pltpu

N_DEV = 8
QROWS = 1024
NC = 8
CM = QROWS // NC


def kernel(x, dy, gamma):
    m, d = x.shape

    def body(x_hbm, dy_hbm, out_ref, xbuf, dybuf, acc_ref, recv_ref,
             copy_sems, send_sems, recv_sems):
        my_id = (
            lax.axis_index("x") * 4 + lax.axis_index("y") * 2 + lax.axis_index("z")
        )
        r = lax.axis_index("y") * 2 + lax.axis_index("z")
        base = r * QROWS

        x_copies = [
            pltpu.make_async_copy(
                x_hbm.at[pl.ds(base + c * CM, CM), :], xbuf.at[c],
                copy_sems.at[0, c],
            )
            for c in range(NC)
        ]
        dy_copies = [
            pltpu.make_async_copy(
                dy_hbm.at[pl.ds(base + c * CM, CM), :], dybuf.at[c],
                copy_sems.at[1, c],
            )
            for c in range(NC)
        ]
        for c in range(NC):
            x_copies[c].start()
            dy_copies[c].start()

        barrier = pltpu.get_barrier_semaphore()
        for k in range(N_DEV):
            @pl.when(my_id != k)
            def _():
                pl.semaphore_signal(
                    barrier, inc=1,
                    device_id=k, device_id_type=pl.DeviceIdType.LOGICAL,
                )
        pl.semaphore_wait(barrier, N_DEV - 1)

        tot = jnp.zeros((2, d), jnp.float32)
        for c in range(NC):
            x_copies[c].wait()
            dy_copies[c].wait()
            xb = xbuf[c]
            dyb = dybuf[c]
            mu = jnp.mean(xb, axis=1, keepdims=True)
            diff = xb - mu
            var = jnp.mean(diff * diff, axis=1, keepdims=True)
            xhat = diff * lax.rsqrt(var + 1e-5)
            pdg = jnp.sum(dyb * xhat, axis=0)[None, :]
            pdb = jnp.sum(dyb, axis=0)[None, :]
            tot = tot + jnp.concatenate([pdg, pdb], axis=0)
        acc_ref[...] = tot

        send_descs = [
            pltpu.make_async_remote_copy(
                src_ref=acc_ref,
                dst_ref=recv_ref.at[my_id],
                send_sem=send_sems.at[k],
                recv_sem=recv_sems.at[my_id],
                device_id=k,
                device_id_type=pl.DeviceIdType.LOGICAL,
            )
            for k in range(N_DEV)
        ]
        recv_descs = [
            pltpu.make_async_remote_copy(
                src_ref=acc_ref,
                dst_ref=recv_ref.at[k],
                send_sem=send_sems.at[k],
                recv_sem=recv_sems.at[k],
                device_id=k,
                device_id_type=pl.DeviceIdType.LOGICAL,
            )
            for k in range(N_DEV)
        ]

        for k in range(N_DEV):
            @pl.when(my_id != k)
            def _(k=k):
                send_descs[k].start()

        total = acc_ref[...]
        for k in range(N_DEV):
            @pl.when(my_id != k)
            def _(k=k):
                recv_descs[k].wait_recv()
            total = total + jnp.where(my_id != k, recv_ref[k], 0.0)
        out_ref[...] = total

        for k in range(N_DEV):
            @pl.when(my_id != k)
            def _(k=k):
                send_descs[k].wait_send()

    return pl.pallas_call(
        body,
        in_specs=[
            pl.BlockSpec(memory_space=pltpu.MemorySpace.HBM),
            pl.BlockSpec(memory_space=pltpu.MemorySpace.HBM),
        ],
        out_specs=pl.BlockSpec(memory_space=pltpu.VMEM),
        out_shape=jax.ShapeDtypeStruct((2, d), jnp.float32),
        scratch_shapes=[
            pltpu.VMEM((NC, CM, d), jnp.float32),
            pltpu.VMEM((NC, CM, d), jnp.float32),
            pltpu.VMEM((2, d), jnp.float32),
            pltpu.VMEM((N_DEV, 2, d), jnp.float32),
            pltpu.SemaphoreType.DMA((2, NC)),
            pltpu.SemaphoreType.DMA((N_DEV,)),
            pltpu.SemaphoreType.DMA((N_DEV,)),
        ],
        compiler_params=pltpu.CompilerParams(collective_id=0),
    )(x, dy)
